# Initial kernel scaffold; baseline (speedup 1.0000x reference)
import jax
import jax.numpy as jnp
from jax.experimental import pallas as pl


def kernel(x_obj, x_pred, pair_idxs, W_ev, b_ev, W_ve, b_ve, W_ih, W_hh, b_ih, b_hh):
    raise NotImplementedError("write your pallas kernel here")


# speedup vs baseline: 2.3289x; 2.3289x over previous
import functools

import jax
import jax.numpy as jnp
from jax import lax
from jax.experimental import pallas as pl
from jax.experimental.pallas import tpu as pltpu
from jax.experimental.pallas import tpu_sc as plsc

NUM_CORES = 2
NUM_SUBCORES = 16


def _pad_rows(n):
    m = 8 * NUM_SUBCORES
    return ((n + m - 1) // m) * m


def _pick_chunk(per_tile):
    for ch in (128, 120, 112, 104, 96, 88, 80, 72, 64, 56, 48, 40, 32, 24, 16, 8):
        if per_tile % ch == 0:
            return ch
    return 1


def _sc_mesh():
    return plsc.VectorSubcoreMesh(
        core_axis_name="c", subcore_axis_name="s",
        num_cores=NUM_CORES, num_subcores=NUM_SUBCORES)


def _make_gather(n, r, d):
    per_tile = r // NUM_SUBCORES
    ch = _pick_chunk(per_tile)
    n_ch = per_tile // ch

    @functools.partial(
        pl.kernel, mesh=_sc_mesh(),
        out_type=jax.ShapeDtypeStruct((2, r, d), jnp.float32),
        scratch_types=[
            pltpu.VMEM((ch,), jnp.int32),
            pltpu.VMEM((ch, d), jnp.float32),
            pltpu.SemaphoreType.DMA,
        ],
    )
    def gather_k(table, idx_s, idx_o, out, idx_v, rows_v, sem):
        c = lax.axis_index("c")
        s = lax.axis_index("s")
        tile_base = s * per_tile

        def chunk_loop(idx_ref):
            def body(i, carry):
                base = pl.multiple_of(tile_base + i * ch, 8)
                pltpu.sync_copy(idx_ref.at[pl.ds(base, ch)], idx_v)
                pltpu.async_copy(table.at[idx_v], rows_v, sem).wait()
                pltpu.sync_copy(rows_v, out.at[c, pl.ds(base, ch)])
                return carry
            lax.fori_loop(0, n_ch, body, 0)

        pl.when(c == 0)(lambda: chunk_loop(idx_s))
        pl.when(c == 1)(lambda: chunk_loop(idx_o))

    return gather_k


def _make_scatter(n, r, d):
    per_tile = r // NUM_SUBCORES
    ch = _pick_chunk(per_tile)
    n_ch = per_tile // ch
    npad = _pad_rows(n)
    stripe = npad // NUM_SUBCORES

    @functools.partial(
        pl.kernel, mesh=_sc_mesh(),
        out_type=jax.ShapeDtypeStruct((2, npad, d), jnp.float32),
        scratch_types=[
            pltpu.VMEM((ch,), jnp.int32),
            pltpu.VMEM((ch, d), jnp.float32),
            pltpu.VMEM_SHARED((npad, d), jnp.float32),
        ],
    )
    def scatter_k(msg2, idx_s, idx_o, zeros_nd, out, idx_v, msg_v, acc):
        c = lax.axis_index("c")
        s = lax.axis_index("s")
        pl.when(s == 0)(lambda: pltpu.sync_copy(zeros_nd, acc))
        plsc.subcore_barrier()

        tile_base = s * per_tile

        def chunk_loop(idx_ref):
            def body(i, carry):
                base = pl.multiple_of(tile_base + i * ch, 8)
                pltpu.sync_copy(idx_ref.at[pl.ds(base, ch)], idx_v)
                pltpu.sync_copy(msg2.at[c, pl.ds(base, ch)], msg_v)
                pltpu.sync_copy(msg_v, acc.at[idx_v], add=True)
                return carry
            lax.fori_loop(0, n_ch, body, 0)

        pl.when(c == 0)(lambda: chunk_loop(idx_s))
        pl.when(c == 1)(lambda: chunk_loop(idx_o))

        plsc.subcore_barrier()
        pl.when(s == 0)(lambda: pltpu.sync_copy(acc, out.at[c]))

    return scatter_k


def _edge_body(ab, p, wev, bev, wve, bve, wih, whh, bih, bhh, msg, pnew):
    d = p.shape[-1]
    a = ab[0]
    b = ab[1]
    x = p[...]
    wev1 = wev[0:d, :]
    wev2 = wev[d:2 * d, :]
    wve1 = wve[0:d, :]
    wve2 = wve[d:2 * d, :]
    dot = functools.partial(jnp.dot, preferred_element_type=jnp.float32)

    g2 = dot(x, wev2) + bev[...]
    gs = jax.nn.sigmoid(dot(a, wev1) + g2)
    go = jax.nn.sigmoid(dot(b, wev1) + g2)
    msg[0] = gs * x
    msg[1] = go * x

    pv = dot(x, wve1) + bve[...]
    ps = jax.nn.sigmoid(pv + dot(a, wve2)) * a
    po = jax.nn.sigmoid(pv + dot(b, wve2)) * b
    gin = ps + po

    gi = dot(gin, wih[...]) + bih[...]
    gh = dot(x, whh[...]) + bhh[...]
    rr = jax.nn.sigmoid(gi[:, 0:d] + gh[:, 0:d])
    zz = jax.nn.sigmoid(gi[:, d:2 * d] + gh[:, d:2 * d])
    nn = jnp.tanh(gi[:, 2 * d:3 * d] + rr * gh[:, 2 * d:3 * d])
    pnew[...] = (1.0 - zz) * nn + zz * x


def _make_edge(r, d, blk):
    grid = r // blk
    full = lambda i: (0, 0)
    return pl.pallas_call(
        _edge_body,
        grid=(grid,),
        in_specs=[
            pl.BlockSpec((2, blk, d), lambda i: (0, i, 0)),
            pl.BlockSpec((blk, d), lambda i: (i, 0)),
            pl.BlockSpec((2 * d, d), full),
            pl.BlockSpec((1, d), full),
            pl.BlockSpec((2 * d, d), full),
            pl.BlockSpec((1, d), full),
            pl.BlockSpec((d, 3 * d), full),
            pl.BlockSpec((d, 3 * d), full),
            pl.BlockSpec((1, 3 * d), full),
            pl.BlockSpec((1, 3 * d), full),
        ],
        out_specs=[
            pl.BlockSpec((2, blk, d), lambda i: (0, i, 0)),
            pl.BlockSpec((blk, d), lambda i: (i, 0)),
        ],
        out_shape=[
            jax.ShapeDtypeStruct((2, r, d), jnp.float32),
            jax.ShapeDtypeStruct((r, d), jnp.float32),
        ],
        compiler_params=pltpu.CompilerParams(
            dimension_semantics=("arbitrary",)),
    )


def _node_body(acc, cnt, xo, wih, whh, bih, bhh, xnew):
    d = xo.shape[-1]
    inv_s = 1.0 / jnp.maximum(cnt[0][:, 0:1], 1.0)
    inv_o = 1.0 / jnp.maximum(cnt[1][:, 0:1], 1.0)
    gin = acc[0] * inv_s + acc[1] * inv_o
    h = xo[...]
    dot = functools.partial(jnp.dot, preferred_element_type=jnp.float32)
    gi = dot(gin, wih[...]) + bih[...]
    gh = dot(h, whh[...]) + bhh[...]
    rr = jax.nn.sigmoid(gi[:, 0:d] + gh[:, 0:d])
    zz = jax.nn.sigmoid(gi[:, d:2 * d] + gh[:, d:2 * d])
    nn = jnp.tanh(gi[:, 2 * d:3 * d] + rr * gh[:, 2 * d:3 * d])
    xnew[...] = (1.0 - zz) * nn + zz * h


def _make_node(n, d, w, blk):
    grid = n // blk
    full = lambda i: (0, 0)
    return pl.pallas_call(
        _node_body,
        grid=(grid,),
        in_specs=[
            pl.BlockSpec((2, blk, d), lambda i: (0, i, 0)),
            pl.BlockSpec((2, blk, w), lambda i: (0, i, 0)),
            pl.BlockSpec((blk, d), lambda i: (i, 0)),
            pl.BlockSpec((d, 3 * d), full),
            pl.BlockSpec((d, 3 * d), full),
            pl.BlockSpec((1, 3 * d), full),
            pl.BlockSpec((1, 3 * d), full),
        ],
        out_specs=pl.BlockSpec((blk, d), lambda i: (i, 0)),
        out_shape=jax.ShapeDtypeStruct((n, d), jnp.float32),
        compiler_params=pltpu.CompilerParams(
            dimension_semantics=("arbitrary",)),
    )


def _pick_blk(total, cap):
    for blk in (cap, 4000, 3200, 2560, 2000, 1600, 1280, 1000, 800, 640,
                500, 400, 320, 250, 200, 160, 128, 100, 80, 64, 40, 32, 16, 8):
        if blk <= cap and total % blk == 0:
            return blk
    return 8


T_ITERS = 2


def kernel(x_obj, x_pred, pair_idxs, W_ev, b_ev, W_ve, b_ve, W_ih, W_hh, b_ih, b_hh):
    n, d = x_obj.shape
    r = x_pred.shape[0]

    idx_s = pair_idxs[:, 0].astype(jnp.int32)
    idx_o = pair_idxs[:, 1].astype(jnp.int32)

    gather = _make_gather(n, r, d)
    scatter = _make_scatter(n, r, d)
    edge = _make_edge(r, d, _pick_blk(r, 2000))
    node = _make_node(n, d, d, _pick_blk(n, 2000))

    npad = _pad_rows(n)
    zeros_nd = jnp.zeros((npad, d), jnp.float32)

    bev = b_ev.reshape(1, d)
    bve = b_ve.reshape(1, d)
    bih = b_ih.reshape(1, 3 * d)
    bhh = b_hh.reshape(1, 3 * d)

    ones2 = jnp.ones((2, r, d), jnp.float32)
    cnt2 = scatter(ones2, idx_s, idx_o, zeros_nd)

    def step(carry, _):
        xo, xp = carry
        ab = gather(xo, idx_s, idx_o)
        msg2, xp_new = edge(ab, xp, W_ev, bev, W_ve, bve,
                            W_ih, W_hh, bih, bhh)
        acc = scatter(msg2, idx_s, idx_o, zeros_nd)
        xo_new = node(acc, cnt2, xo, W_ih, W_hh, bih, bhh)
        return (xo_new, xp_new), 0

    (x_obj, x_pred), _ = lax.scan(step, (x_obj, x_pred), None, length=T_ITERS)
    return (x_obj, x_pred)


# speedup vs baseline: 2.7360x; 1.1748x over previous
import functools

import jax
import jax.numpy as jnp
from jax import lax
from jax.experimental import pallas as pl
from jax.experimental.pallas import tpu as pltpu
from jax.experimental.pallas import tpu_sc as plsc

NUM_CORES = 2
NUM_SUBCORES = 16


def _pick_chunk(per_tile):
    for ch in (128, 120, 112, 104, 96, 88, 80, 72, 64, 56, 48, 40, 32, 24, 16, 8):
        if per_tile % ch == 0:
            return ch
    return 1


def _sc_mesh():
    return plsc.VectorSubcoreMesh(
        core_axis_name="c", subcore_axis_name="s",
        num_cores=NUM_CORES, num_subcores=NUM_SUBCORES)


def _make_gather(n, r, d):
    per_tile = r // NUM_SUBCORES
    ch = _pick_chunk(per_tile)
    n_ch = per_tile // ch
    assert n_ch % 2 == 0

    @functools.partial(
        pl.kernel, mesh=_sc_mesh(),
        out_type=jax.ShapeDtypeStruct((2, r, d), jnp.float32),
        scratch_types=[
            pltpu.VMEM((2, 1, ch), jnp.int32),
            pltpu.VMEM((ch, d), jnp.float32),
            pltpu.VMEM((ch, d), jnp.float32),
            pltpu.SemaphoreType.DMA,
            pltpu.SemaphoreType.DMA,
            pltpu.SemaphoreType.DMA,
            pltpu.SemaphoreType.DMA,
            pltpu.SemaphoreType.DMA,
            pltpu.SemaphoreType.DMA,
        ],
    )
    def gather_k(table, idx4_s, idx4_o, out, idxb, rows0, rows1,
                 si0, si1, sg0, sg1, sw0, sw1):
        c = lax.axis_index("c")
        s = lax.axis_index("s")
        tile_base = s * per_tile
        rows = (rows0, rows1)
        si = (si0, si1)
        sg = (sg0, sg1)
        sw = (sw0, sw1)

        def run(idx4):
            pltpu.sync_copy(idx4.at[s, 0], idxb.at[0])

            def body(t, carry):
                for par in range(2):
                    b = 2 * t + par
                    base = pl.multiple_of(tile_base + b * ch, 8)
                    pl.when(b > 0)(lambda par=par: pltpu.make_async_copy(
                        idx4.at[s, 0], idxb.at[par], si[par]).wait())
                    def prefetch(par=par, b=b):
                        pltpu.async_copy(idx4.at[s, b + 1],
                                         idxb.at[1 - par], si[1 - par])
                    pl.when(b + 1 < n_ch)(prefetch)
                    pl.when(b >= 2)(lambda par=par, base=base:
                                    pltpu.make_async_copy(
                                        rows[par],
                                        out.at[c, pl.ds(base, ch)],
                                        sw[par]).wait())
                    pltpu.async_copy(table.at[idxb.at[par, 0]], rows[par],
                                     sg[par]).wait()
                    pltpu.async_copy(rows[par], out.at[c, pl.ds(base, ch)],
                                     sw[par])
                return carry

            lax.fori_loop(0, n_ch // 2, body, 0)
            for par in range(2):
                pltpu.make_async_copy(
                    rows[par], out.at[c, pl.ds(tile_base, ch)],
                    sw[par]).wait()

        pl.when(c == 0)(lambda: run(idx4_s))
        pl.when(c == 1)(lambda: run(idx4_o))

    return gather_k


def _make_scatter(n, r, d):
    per_tile = r // NUM_SUBCORES
    ch = _pick_chunk(per_tile)
    n_ch = per_tile // ch
    assert n_ch % 2 == 0

    @functools.partial(
        pl.kernel, mesh=_sc_mesh(),
        out_type=jax.ShapeDtypeStruct((2, n, d), jnp.float32),
        scratch_types=[
            pltpu.VMEM((2, 1, ch), jnp.int32),
            pltpu.VMEM((ch, d), jnp.float32),
            pltpu.VMEM((ch, d), jnp.float32),
            pltpu.VMEM_SHARED((n, d), jnp.float32),
            pltpu.SemaphoreType.DMA,
            pltpu.SemaphoreType.DMA,
            pltpu.SemaphoreType.DMA,
            pltpu.SemaphoreType.DMA,
        ],
    )
    def scatter_k(msg2, idx4_s, idx4_o, zeros_nd, out, idxb, buf0, buf1,
                  acc, si0, si1, sa0, sa1):
        c = lax.axis_index("c")
        s = lax.axis_index("s")
        pl.when(s == 0)(lambda: pltpu.sync_copy(zeros_nd, acc))
        plsc.subcore_barrier()

        tile_base = s * per_tile
        bufs = (buf0, buf1)
        si = (si0, si1)
        sa = (sa0, sa1)

        def drain_add(par):
            pltpu.make_async_copy(
                bufs[par], acc.at[idxb.at[par, 0]], sa[par]).wait()

        def run(idx4):
            pltpu.sync_copy(idx4.at[s, 0], idxb.at[0])

            def body(t, carry):
                for par in range(2):
                    b = 2 * t + par
                    base = pl.multiple_of(tile_base + b * ch, 8)
                    pl.when(b > 0)(lambda par=par: pltpu.make_async_copy(
                        idx4.at[s, 0], idxb.at[par], si[par]).wait())
                    pl.when(b >= 1)(lambda par=par: drain_add(1 - par))
                    def prefetch(par=par, b=b):
                        pltpu.async_copy(idx4.at[s, b + 1],
                                         idxb.at[1 - par], si[1 - par])
                    pl.when(b + 1 < n_ch)(prefetch)
                    pltpu.sync_copy(msg2.at[c, pl.ds(base, ch)], bufs[par])
                    pltpu.async_copy(bufs[par], acc.at[idxb.at[par, 0]],
                                     sa[par], add=True)
                return carry

            lax.fori_loop(0, n_ch // 2, body, 0)
            drain_add(1)

        pl.when(c == 0)(lambda: run(idx4_s))
        pl.when(c == 1)(lambda: run(idx4_o))

        plsc.subcore_barrier()
        pl.when(s == 0)(lambda: pltpu.sync_copy(acc, out.at[c]))

    return scatter_k


def _edge_body(ab, p, wev, bev, wve, bve, wih, whh, bih, bhh, msg, pnew):
    d = p.shape[-1]
    a = ab[0]
    b = ab[1]
    x = p[...]
    bf = jnp.bfloat16
    a16 = a.astype(bf)
    b16 = b.astype(bf)
    x16 = x.astype(bf)
    wev16 = wev[...].astype(bf)
    wve16 = wve[...].astype(bf)
    wev1 = wev16[0:d, :]
    wev2 = wev16[d:2 * d, :]
    wve1 = wve16[0:d, :]
    wve2 = wve16[d:2 * d, :]
    dot = functools.partial(jnp.dot, preferred_element_type=jnp.float32)

    g2 = dot(x16, wev2) + bev[...]
    gs = jax.nn.sigmoid(dot(a16, wev1) + g2)
    go = jax.nn.sigmoid(dot(b16, wev1) + g2)
    msg[0] = gs * x
    msg[1] = go * x

    pv = dot(x16, wve1) + bve[...]
    ps = jax.nn.sigmoid(pv + dot(a16, wve2)) * a
    po = jax.nn.sigmoid(pv + dot(b16, wve2)) * b
    gin = ps + po

    gi = dot(gin.astype(bf), wih[...].astype(bf)) + bih[...]
    gh = dot(x16, whh[...].astype(bf)) + bhh[...]
    rr = jax.nn.sigmoid(gi[:, 0:d] + gh[:, 0:d])
    zz = jax.nn.sigmoid(gi[:, d:2 * d] + gh[:, d:2 * d])
    nn = jnp.tanh(gi[:, 2 * d:3 * d] + rr * gh[:, 2 * d:3 * d])
    pnew[...] = (1.0 - zz) * nn + zz * x


def _make_edge(r, d, blk):
    grid = r // blk
    full = lambda i: (0, 0)
    return pl.pallas_call(
        _edge_body,
        grid=(grid,),
        in_specs=[
            pl.BlockSpec((2, blk, d), lambda i: (0, i, 0)),
            pl.BlockSpec((blk, d), lambda i: (i, 0)),
            pl.BlockSpec((2 * d, d), full),
            pl.BlockSpec((1, d), full),
            pl.BlockSpec((2 * d, d), full),
            pl.BlockSpec((1, d), full),
            pl.BlockSpec((d, 3 * d), full),
            pl.BlockSpec((d, 3 * d), full),
            pl.BlockSpec((1, 3 * d), full),
            pl.BlockSpec((1, 3 * d), full),
        ],
        out_specs=[
            pl.BlockSpec((2, blk, d), lambda i: (0, i, 0)),
            pl.BlockSpec((blk, d), lambda i: (i, 0)),
        ],
        out_shape=[
            jax.ShapeDtypeStruct((2, r, d), jnp.float32),
            jax.ShapeDtypeStruct((r, d), jnp.float32),
        ],
        compiler_params=pltpu.CompilerParams(
            dimension_semantics=("arbitrary",)),
    )


def _node_body(acc, cnt, xo, wih, whh, bih, bhh, xnew):
    d = xo.shape[-1]
    inv_s = 1.0 / jnp.maximum(cnt[0][:, 0:1], 1.0)
    inv_o = 1.0 / jnp.maximum(cnt[1][:, 0:1], 1.0)
    gin = acc[0] * inv_s + acc[1] * inv_o
    h = xo[...]
    dot = functools.partial(jnp.dot, preferred_element_type=jnp.float32)
    gi = dot(gin, wih[...]) + bih[...]
    gh = dot(h, whh[...]) + bhh[...]
    rr = jax.nn.sigmoid(gi[:, 0:d] + gh[:, 0:d])
    zz = jax.nn.sigmoid(gi[:, d:2 * d] + gh[:, d:2 * d])
    nn = jnp.tanh(gi[:, 2 * d:3 * d] + rr * gh[:, 2 * d:3 * d])
    xnew[...] = (1.0 - zz) * nn + zz * h


def _make_node(n, d, w, blk):
    grid = n // blk
    full = lambda i: (0, 0)
    return pl.pallas_call(
        _node_body,
        grid=(grid,),
        in_specs=[
            pl.BlockSpec((2, blk, d), lambda i: (0, i, 0)),
            pl.BlockSpec((2, blk, w), lambda i: (0, i, 0)),
            pl.BlockSpec((blk, d), lambda i: (i, 0)),
            pl.BlockSpec((d, 3 * d), full),
            pl.BlockSpec((d, 3 * d), full),
            pl.BlockSpec((1, 3 * d), full),
            pl.BlockSpec((1, 3 * d), full),
        ],
        out_specs=pl.BlockSpec((blk, d), lambda i: (i, 0)),
        out_shape=jax.ShapeDtypeStruct((n, d), jnp.float32),
        compiler_params=pltpu.CompilerParams(
            dimension_semantics=("arbitrary",)),
    )


def _pick_blk(total, cap):
    for blk in (cap, 4000, 3200, 2560, 2000, 1600, 1280, 1000, 800, 640,
                500, 400, 320, 250, 200, 160, 128, 100, 80, 64, 40, 32, 16, 8):
        if blk <= cap and total % blk == 0:
            return blk
    return 8


T_ITERS = 2


def kernel(x_obj, x_pred, pair_idxs, W_ev, b_ev, W_ve, b_ve, W_ih, W_hh, b_ih, b_hh):
    n, d = x_obj.shape
    r = x_pred.shape[0]

    per_tile = r // NUM_SUBCORES
    ch = _pick_chunk(per_tile)
    n_ch = per_tile // ch
    idx_s = pair_idxs[:, 0].astype(jnp.int32).reshape(
        NUM_SUBCORES, n_ch, 1, ch)
    idx_o = pair_idxs[:, 1].astype(jnp.int32).reshape(
        NUM_SUBCORES, n_ch, 1, ch)

    gather = _make_gather(n, r, d)
    scatter = _make_scatter(n, r, d)
    edge = _make_edge(r, d, _pick_blk(r, 2000))
    node = _make_node(n, d, d, _pick_blk(n, 2000))

    zeros_nd = jnp.zeros((n, d), jnp.float32)

    bev = b_ev.reshape(1, d)
    bve = b_ve.reshape(1, d)
    bih = b_ih.reshape(1, 3 * d)
    bhh = b_hh.reshape(1, 3 * d)

    ones2 = jnp.ones((2, r, d), jnp.float32)
    cnt2 = scatter(ones2, idx_s, idx_o, zeros_nd)

    def step(carry, _):
        xo, xp = carry
        ab = gather(xo, idx_s, idx_o)
        msg2, xp_new = edge(ab, xp, W_ev, bev, W_ve, bve,
                            W_ih, W_hh, bih, bhh)
        acc = scatter(msg2, idx_s, idx_o, zeros_nd)
        xo_new = node(acc, cnt2, xo, W_ih, W_hh, bih, bhh)
        return (xo_new, xp_new), 0

    (x_obj, x_pred), _ = lax.scan(step, (x_obj, x_pred), None, length=T_ITERS)
    return (x_obj, x_pred)


# speedup vs baseline: 3.0158x; 1.1023x over previous
import functools

import jax
import jax.numpy as jnp
from jax import lax
from jax.experimental import pallas as pl
from jax.experimental.pallas import tpu as pltpu
from jax.experimental.pallas import tpu_sc as plsc

NUM_CORES = 2
NUM_SUBCORES = 16


def _pick_chunk(per_tile):
    for ch in (128, 120, 112, 104, 96, 88, 80, 72, 64, 56, 48, 40, 32, 24, 16, 8):
        if per_tile % ch == 0:
            return ch
    return 1


def _sc_mesh():
    return plsc.VectorSubcoreMesh(
        core_axis_name="c", subcore_axis_name="s",
        num_cores=NUM_CORES, num_subcores=NUM_SUBCORES)


def _make_gather(n, r, d):
    per_tile = r // NUM_SUBCORES
    ch = _pick_chunk(per_tile)
    n_ch = per_tile // ch
    assert n_ch % 2 == 0

    @functools.partial(
        pl.kernel, mesh=_sc_mesh(),
        out_type=jax.ShapeDtypeStruct((2, r, d), jnp.float32),
        scratch_types=[
            pltpu.VMEM((2, 1, ch), jnp.int32),
            pltpu.VMEM((ch, d), jnp.float32),
            pltpu.VMEM((ch, d), jnp.float32),
            pltpu.SemaphoreType.DMA,
            pltpu.SemaphoreType.DMA,
            pltpu.SemaphoreType.DMA,
            pltpu.SemaphoreType.DMA,
            pltpu.SemaphoreType.DMA,
            pltpu.SemaphoreType.DMA,
        ],
    )
    def gather_k(table, idx4_s, idx4_o, out, idxb, rows0, rows1,
                 si0, si1, sg0, sg1, sw0, sw1):
        c = lax.axis_index("c")
        s = lax.axis_index("s")
        tile_base = s * per_tile
        rows = (rows0, rows1)
        si = (si0, si1)
        sg = (sg0, sg1)
        sw = (sw0, sw1)

        def run(idx4):
            pltpu.sync_copy(idx4.at[s, 0], idxb.at[0])
            pltpu.async_copy(table.at[idxb.at[0, 0]], rows[0], sg[0])
            pltpu.async_copy(idx4.at[s, 1], idxb.at[1], si[1])

            def wait_idx(par):
                pltpu.make_async_copy(
                    idx4.at[s, 0], idxb.at[par], si[par]).wait()

            def wait_gather(par):
                pltpu.make_async_copy(
                    table.at[idxb.at[par, 0]], rows[par], sg[par]).wait()

            def wait_wb(par):
                pltpu.make_async_copy(
                    rows[par], out.at[c, pl.ds(tile_base, ch)],
                    sw[par]).wait()

            def body(t, carry):
                for par in range(2):
                    b = 2 * t + par
                    base = pl.multiple_of(tile_base + b * ch, 8)
                    prev_base = pl.multiple_of(tile_base + (b - 1) * ch, 8)
                    pl.when(b >= 1)(lambda par=par: wait_idx(par))
                    pl.when(b >= 1)(lambda par=par: wait_gather(1 - par))

                    def wb_prev(par=par, prev_base=prev_base):
                        pltpu.async_copy(
                            rows[1 - par],
                            out.at[c, pl.ds(prev_base, ch)], sw[1 - par])
                    pl.when(b >= 1)(wb_prev)
                    pl.when(b >= 2)(lambda par=par: wait_wb(par))

                    def fire(par=par):
                        pltpu.async_copy(table.at[idxb.at[par, 0]],
                                         rows[par], sg[par])
                    pl.when(b >= 1)(fire)

                    def prefetch(par=par, b=b):
                        pltpu.async_copy(idx4.at[s, b + 1],
                                         idxb.at[1 - par], si[1 - par])
                    pl.when(jnp.logical_and(b >= 1, b + 1 < n_ch))(prefetch)
                return carry

            lax.fori_loop(0, n_ch // 2, body, 0)
            wait_gather(1)
            last_base = pl.multiple_of(tile_base + (n_ch - 1) * ch, 8)
            pltpu.async_copy(rows[1], out.at[c, pl.ds(last_base, ch)], sw[1])
            wait_wb(0)
            wait_wb(1)

        pl.when(c == 0)(lambda: run(idx4_s))
        pl.when(c == 1)(lambda: run(idx4_o))

    return gather_k


def _make_scatter(n, r, d):
    per_tile = r // NUM_SUBCORES
    ch = _pick_chunk(per_tile)
    n_ch = per_tile // ch
    assert n_ch % 2 == 0

    @functools.partial(
        pl.kernel, mesh=_sc_mesh(),
        out_type=jax.ShapeDtypeStruct((2, n, d), jnp.float32),
        scratch_types=[
            pltpu.VMEM((2, 1, ch), jnp.int32),
            pltpu.VMEM((ch, d), jnp.float32),
            pltpu.VMEM((ch, d), jnp.float32),
            pltpu.VMEM_SHARED((n, d), jnp.float32),
            pltpu.SemaphoreType.DMA,
            pltpu.SemaphoreType.DMA,
            pltpu.SemaphoreType.DMA,
            pltpu.SemaphoreType.DMA,
            pltpu.SemaphoreType.DMA,
            pltpu.SemaphoreType.DMA,
        ],
    )
    def scatter_k(msg2, idx4_s, idx4_o, zeros_nd, out, idxb, buf0, buf1,
                  acc, si0, si1, sa0, sa1, sm0, sm1):
        c = lax.axis_index("c")
        s = lax.axis_index("s")
        pl.when(s == 0)(lambda: pltpu.sync_copy(zeros_nd, acc))
        plsc.subcore_barrier()

        tile_base = s * per_tile
        bufs = (buf0, buf1)
        si = (si0, si1)
        sa = (sa0, sa1)
        sm = (sm0, sm1)

        def drain_add(par):
            pltpu.make_async_copy(
                bufs[par], acc.at[idxb.at[par, 0]], sa[par]).wait()

        def run(idx4):
            pltpu.sync_copy(idx4.at[s, 0], idxb.at[0])
            pltpu.sync_copy(msg2.at[c, pl.ds(tile_base, ch)], bufs[0])
            pltpu.async_copy(idx4.at[s, 1], idxb.at[1], si[1])
            pltpu.async_copy(msg2.at[c, pl.ds(
                pl.multiple_of(tile_base + ch, 8), ch)], bufs[1], sm[1])

            def wait_idx(par):
                pltpu.make_async_copy(
                    idx4.at[s, 0], idxb.at[par], si[par]).wait()

            def wait_msg(par):
                pltpu.make_async_copy(
                    msg2.at[c, pl.ds(tile_base, ch)], bufs[par],
                    sm[par]).wait()

            def body(t, carry):
                for par in range(2):
                    b = 2 * t + par
                    pl.when(b >= 1)(lambda par=par: wait_idx(par))
                    pl.when(b >= 1)(lambda par=par: wait_msg(par))
                    pl.when(b >= 1)(lambda par=par: drain_add(1 - par))
                    pltpu.async_copy(bufs[par], acc.at[idxb.at[par, 0]],
                                     sa[par], add=True)

                    def prefetch(par=par, b=b):
                        nbase = pl.multiple_of(tile_base + (b + 1) * ch, 8)
                        pltpu.async_copy(idx4.at[s, b + 1],
                                         idxb.at[1 - par], si[1 - par])
                        pltpu.async_copy(msg2.at[c, pl.ds(nbase, ch)],
                                         bufs[1 - par], sm[1 - par])
                    pl.when(jnp.logical_and(b >= 1, b + 1 < n_ch))(prefetch)
                return carry

            lax.fori_loop(0, n_ch // 2, body, 0)
            drain_add(1)

        pl.when(c == 0)(lambda: run(idx4_s))
        pl.when(c == 1)(lambda: run(idx4_o))

        plsc.subcore_barrier()
        pl.when(s == 0)(lambda: pltpu.sync_copy(acc, out.at[c]))

    return scatter_k


def _edge_body(ab, p, wev, bev, wve, bve, wih, whh, bih, bhh, msg, pnew):
    d = p.shape[-1]
    a = ab[0]
    b = ab[1]
    x = p[...]
    bf = jnp.bfloat16
    a16 = a.astype(bf)
    b16 = b.astype(bf)
    x16 = x.astype(bf)
    wev16 = wev[...].astype(bf)
    wve16 = wve[...].astype(bf)
    wev1 = wev16[0:d, :]
    wev2 = wev16[d:2 * d, :]
    wve1 = wve16[0:d, :]
    wve2 = wve16[d:2 * d, :]
    dot = functools.partial(jnp.dot, preferred_element_type=jnp.float32)

    g2 = dot(x16, wev2) + bev[...]
    gs = jax.nn.sigmoid(dot(a16, wev1) + g2)
    go = jax.nn.sigmoid(dot(b16, wev1) + g2)
    msg[0] = gs * x
    msg[1] = go * x

    pv = dot(x16, wve1) + bve[...]
    ps = jax.nn.sigmoid(pv + dot(a16, wve2)) * a
    po = jax.nn.sigmoid(pv + dot(b16, wve2)) * b
    gin = ps + po

    gi = dot(gin.astype(bf), wih[...].astype(bf)) + bih[...]
    gh = dot(x16, whh[...].astype(bf)) + bhh[...]
    rr = jax.nn.sigmoid(gi[:, 0:d] + gh[:, 0:d])
    zz = jax.nn.sigmoid(gi[:, d:2 * d] + gh[:, d:2 * d])
    nn = jnp.tanh(gi[:, 2 * d:3 * d] + rr * gh[:, 2 * d:3 * d])
    pnew[...] = (1.0 - zz) * nn + zz * x


def _make_edge(r, d, blk):
    grid = r // blk
    full = lambda i: (0, 0)
    return pl.pallas_call(
        _edge_body,
        grid=(grid,),
        in_specs=[
            pl.BlockSpec((2, blk, d), lambda i: (0, i, 0)),
            pl.BlockSpec((blk, d), lambda i: (i, 0)),
            pl.BlockSpec((2 * d, d), full),
            pl.BlockSpec((1, d), full),
            pl.BlockSpec((2 * d, d), full),
            pl.BlockSpec((1, d), full),
            pl.BlockSpec((d, 3 * d), full),
            pl.BlockSpec((d, 3 * d), full),
            pl.BlockSpec((1, 3 * d), full),
            pl.BlockSpec((1, 3 * d), full),
        ],
        out_specs=[
            pl.BlockSpec((2, blk, d), lambda i: (0, i, 0)),
            pl.BlockSpec((blk, d), lambda i: (i, 0)),
        ],
        out_shape=[
            jax.ShapeDtypeStruct((2, r, d), jnp.float32),
            jax.ShapeDtypeStruct((r, d), jnp.float32),
        ],
        compiler_params=pltpu.CompilerParams(
            dimension_semantics=("arbitrary",)),
    )


def _node_body(acc, cnt, xo, wih, whh, bih, bhh, xnew):
    d = xo.shape[-1]
    inv_s = 1.0 / jnp.maximum(cnt[0][:, 0:1], 1.0)
    inv_o = 1.0 / jnp.maximum(cnt[1][:, 0:1], 1.0)
    gin = acc[0] * inv_s + acc[1] * inv_o
    h = xo[...]
    dot = functools.partial(jnp.dot, preferred_element_type=jnp.float32)
    gi = dot(gin, wih[...]) + bih[...]
    gh = dot(h, whh[...]) + bhh[...]
    rr = jax.nn.sigmoid(gi[:, 0:d] + gh[:, 0:d])
    zz = jax.nn.sigmoid(gi[:, d:2 * d] + gh[:, d:2 * d])
    nn = jnp.tanh(gi[:, 2 * d:3 * d] + rr * gh[:, 2 * d:3 * d])
    xnew[...] = (1.0 - zz) * nn + zz * h


def _make_node(n, d, w, blk):
    grid = n // blk
    full = lambda i: (0, 0)
    return pl.pallas_call(
        _node_body,
        grid=(grid,),
        in_specs=[
            pl.BlockSpec((2, blk, d), lambda i: (0, i, 0)),
            pl.BlockSpec((2, blk, w), lambda i: (0, i, 0)),
            pl.BlockSpec((blk, d), lambda i: (i, 0)),
            pl.BlockSpec((d, 3 * d), full),
            pl.BlockSpec((d, 3 * d), full),
            pl.BlockSpec((1, 3 * d), full),
            pl.BlockSpec((1, 3 * d), full),
        ],
        out_specs=pl.BlockSpec((blk, d), lambda i: (i, 0)),
        out_shape=jax.ShapeDtypeStruct((n, d), jnp.float32),
        compiler_params=pltpu.CompilerParams(
            dimension_semantics=("arbitrary",)),
    )


def _pick_blk(total, cap):
    for blk in (cap, 4000, 3200, 2560, 2000, 1600, 1280, 1000, 800, 640,
                500, 400, 320, 250, 200, 160, 128, 100, 80, 64, 40, 32, 16, 8):
        if blk <= cap and total % blk == 0:
            return blk
    return 8


T_ITERS = 2


def kernel(x_obj, x_pred, pair_idxs, W_ev, b_ev, W_ve, b_ve, W_ih, W_hh, b_ih, b_hh):
    n, d = x_obj.shape
    r = x_pred.shape[0]

    per_tile = r // NUM_SUBCORES
    ch = _pick_chunk(per_tile)
    n_ch = per_tile // ch
    idx_s = pair_idxs[:, 0].astype(jnp.int32).reshape(
        NUM_SUBCORES, n_ch, 1, ch)
    idx_o = pair_idxs[:, 1].astype(jnp.int32).reshape(
        NUM_SUBCORES, n_ch, 1, ch)

    gather = _make_gather(n, r, d)
    scatter = _make_scatter(n, r, d)
    edge = _make_edge(r, d, _pick_blk(r, 2000))
    node = _make_node(n, d, d, _pick_blk(n, 2000))

    zeros_nd = jnp.zeros((n, d), jnp.float32)

    bev = b_ev.reshape(1, d)
    bve = b_ve.reshape(1, d)
    bih = b_ih.reshape(1, 3 * d)
    bhh = b_hh.reshape(1, 3 * d)

    ones2 = jnp.ones((2, r, d), jnp.float32)
    cnt2 = scatter(ones2, idx_s, idx_o, zeros_nd)

    def step(carry, _):
        xo, xp = carry
        ab = gather(xo, idx_s, idx_o)
        msg2, xp_new = edge(ab, xp, W_ev, bev, W_ve, bve,
                            W_ih, W_hh, bih, bhh)
        acc = scatter(msg2, idx_s, idx_o, zeros_nd)
        xo_new = node(acc, cnt2, xo, W_ih, W_hh, bih, bhh)
        return (xo_new, xp_new), 0

    (x_obj, x_pred), _ = lax.scan(step, (x_obj, x_pred), None, length=T_ITERS)
    return (x_obj, x_pred)
